# baseline (device time: 114512 ns/iter reference)
import functools

import jax

try:
    jax.config.update("jax_compilation_cache_dir", "/tmp/jax_persist_cache")
    jax.config.update("jax_persistent_cache_min_compile_time_secs", 1.0)
except Exception:
    pass

import jax.numpy as jnp
from jax import lax
from jax.experimental import pallas as pl
from jax.experimental.pallas import tpu as pltpu

N_DEV = 8
NCHUNK = 512
RING = 3
LEAD = RING - 1


def kernel(x, w_mat):
    m_all, mper = x.shape
    kdim, n = w_mat.shape
    assert m_all == N_DEV * mper == kdim
    n_chunks = n // NCHUNK

    def body(
        x_ref,
        w_ref,
        out_ref,
        xb_ref,
        abuf_ref,
        wf32_ref,
        wb_ref,
        ostage_ref,
        send_sems,
        recv_sems,
        wsems,
        osems,
        lsem,
    ):
        my = lax.axis_index("i")

        def wcopy(c, slot):
            return pltpu.make_async_copy(
                w_ref.at[:, pl.ds(c * NCHUNK, NCHUNK)],
                wf32_ref.at[slot],
                wsems.at[slot],
            )

        def convert(c):
            wcopy(c, c % 2).wait()
            wb_ref[c % RING, :, :] = wf32_ref[c % 2, :, :].astype(jnp.bfloat16)

        wcopy(0, 0).start()

        xb_ref[:, :] = x_ref[:, :].astype(jnp.bfloat16)

        barrier = pltpu.get_barrier_semaphore()
        for p in range(1, N_DEV):
            pl.semaphore_signal(
                barrier,
                inc=1,
                device_id=((my + p) % N_DEV,),
                device_id_type=pl.DeviceIdType.MESH,
            )
        pl.semaphore_wait(barrier, N_DEV - 1)

        sends = []
        for p in range(1, N_DEV):
            dst = (my + p) % N_DEV
            rdma = pltpu.make_async_remote_copy(
                src_ref=xb_ref.at[pl.ds(dst * mper, mper), :],
                dst_ref=abuf_ref.at[:, pl.ds(my * mper, mper)],
                send_sem=send_sems.at[p - 1],
                recv_sem=recv_sems.at[p - 1],
                device_id=(dst,),
                device_id_type=pl.DeviceIdType.MESH,
            )
            rdma.start()
            sends.append(rdma)

        local_cp = pltpu.make_async_copy(
            xb_ref.at[pl.ds(my * mper, mper), :],
            abuf_ref.at[:, pl.ds(my * mper, mper)],
            lsem,
        )
        local_cp.start()

        wcopy(1, 1).start()

        for c in range(LEAD):
            convert(c)
            if c + 2 < n_chunks:
                wcopy(c + 2, c % 2).start()

        for p in range(1, N_DEV):
            recv = pltpu.make_async_remote_copy(
                src_ref=xb_ref.at[pl.ds(0, mper), :],
                dst_ref=abuf_ref.at[:, pl.ds(0, mper)],
                send_sem=send_sems.at[p - 1],
                recv_sem=recv_sems.at[p - 1],
                device_id=((my + p) % N_DEV,),
                device_id_type=pl.DeviceIdType.MESH,
            )
            recv.wait_recv()
        local_cp.wait()

        for c in range(n_chunks):
            if c + LEAD < n_chunks:
                convert(c + LEAD)
                if c + LEAD + 2 < n_chunks:
                    wcopy(c + LEAD + 2, (c + LEAD) % 2).start()
            y = jnp.dot(
                abuf_ref[:, :],
                wb_ref[c % RING],
                preferred_element_type=jnp.float32,
            )
            if c >= 2:
                pltpu.make_async_copy(
                    ostage_ref.at[c % 2], out_ref.at[:, pl.ds(0, NCHUNK)],
                    osems.at[c % 2],
                ).wait()
            ostage_ref[c % 2, :, :] = jax.nn.gelu(y, approximate=True)
            pltpu.make_async_copy(
                ostage_ref.at[c % 2],
                out_ref.at[:, pl.ds(c * NCHUNK, NCHUNK)],
                osems.at[c % 2],
            ).start()

        for c in (n_chunks - 2, n_chunks - 1):
            pltpu.make_async_copy(
                ostage_ref.at[c % 2], out_ref.at[:, pl.ds(0, NCHUNK)],
                osems.at[c % 2],
            ).wait()

        for rdma in sends:
            rdma.wait_send()

        @functools.partial(pl.run_scoped, exit_sem=pltpu.SemaphoreType.REGULAR)
        def _(exit_sem):
            for p in range(1, N_DEV):
                pl.semaphore_signal(
                    exit_sem,
                    inc=1,
                    device_id=((my + p) % N_DEV,),
                    device_id_type=pl.DeviceIdType.MESH,
                )
            pl.semaphore_wait(exit_sem, N_DEV - 1)

    return pl.pallas_call(
        body,
        out_shape=jax.ShapeDtypeStruct((mper, n), jnp.float32),
        in_specs=[
            pl.BlockSpec(memory_space=pltpu.MemorySpace.VMEM),
            pl.BlockSpec(memory_space=pltpu.MemorySpace.HBM),
        ],
        out_specs=pl.BlockSpec(memory_space=pltpu.MemorySpace.HBM),
        scratch_shapes=[
            pltpu.VMEM((m_all, mper), jnp.bfloat16),
            pltpu.VMEM((mper, kdim), jnp.bfloat16),
            pltpu.VMEM((2, kdim, NCHUNK), jnp.float32),
            pltpu.VMEM((RING, kdim, NCHUNK), jnp.bfloat16),
            pltpu.VMEM((2, mper, NCHUNK), jnp.float32),
            pltpu.SemaphoreType.DMA((N_DEV - 1,)),
            pltpu.SemaphoreType.DMA((N_DEV - 1,)),
            pltpu.SemaphoreType.DMA((2,)),
            pltpu.SemaphoreType.DMA((2,)),
            pltpu.SemaphoreType.DMA,
        ],
        compiler_params=pltpu.CompilerParams(
            collective_id=0,
            vmem_limit_bytes=100 * 1024 * 1024,
        ),
    )(x, w_mat)


# device time: 100092 ns/iter; 1.1441x vs baseline; 1.1441x over previous
import functools

import jax

try:
    jax.config.update("jax_compilation_cache_dir", "/tmp/jax_persist_cache")
    jax.config.update("jax_persistent_cache_min_compile_time_secs", 1.0)
except Exception:
    pass

import jax.numpy as jnp
from jax import lax
from jax.experimental import pallas as pl
from jax.experimental.pallas import tpu as pltpu

N_DEV = 8
NCHUNK = 4096
NUNIT = 2048
RING = 3


def kernel(x, w_mat):
    m_all, mper = x.shape
    kdim, n = w_mat.shape
    assert m_all == N_DEV * mper == kdim
    n_chunks = n // NCHUNK
    n_steps = N_DEV * n_chunks
    n_units = n_steps * (NCHUNK // NUNIT)
    upc = NCHUNK // NUNIT

    def body(
        x_ref,
        w_ref,
        out_ref,
        xb_ref,
        recv_ref,
        wf32_ref,
        wb_ref,
        acc_ref,
        send_sems,
        recv_sems,
        wsems,
        osems,
    ):
        my = lax.axis_index("i")

        def unit_start(u):
            t, h = u // upc, u % upc
            s, c = t // n_chunks, t % n_chunks
            j = (my - s) % N_DEV
            pltpu.make_async_copy(
                w_ref.at[
                    pl.ds(j * mper, mper),
                    pl.ds(c * NCHUNK + h * NUNIT, NUNIT),
                ],
                wf32_ref.at[u % 2],
                wsems.at[u % 2],
            ).start()

        def unit_convert(u):
            t, h = u // upc, u % upc
            pltpu.make_async_copy(
                w_ref.at[pl.ds(0, mper), pl.ds(0, NUNIT)],
                wf32_ref.at[u % 2],
                wsems.at[u % 2],
            ).wait()
            wb_ref[t % RING, :, pl.ds(h * NUNIT, NUNIT)] = wf32_ref[
                u % 2, :, :
            ].astype(jnp.bfloat16)

        unit_start(0)

        xb_ref[:, :] = x_ref[:, :].astype(jnp.bfloat16)

        barrier = pltpu.get_barrier_semaphore()
        for p in range(1, N_DEV):
            pl.semaphore_signal(
                barrier,
                inc=1,
                device_id=((my + p) % N_DEV,),
                device_id_type=pl.DeviceIdType.MESH,
            )
        pl.semaphore_wait(barrier, N_DEV - 1)

        sends = []
        for p in range(1, N_DEV):
            dst = (my + p) % N_DEV
            rdma = pltpu.make_async_remote_copy(
                src_ref=xb_ref.at[pl.ds(dst * mper, mper), :],
                dst_ref=recv_ref.at[p - 1],
                send_sem=send_sems.at[p - 1],
                recv_sem=recv_sems.at[p - 1],
                device_id=(dst,),
                device_id_type=pl.DeviceIdType.MESH,
            )
            rdma.start()
            sends.append(rdma)

        state = {"started": 1, "conv": 0}

        def pump(target_conv):
            while state["conv"] < min(target_conv, n_units):
                if state["started"] < min(state["conv"] + 2, n_units):
                    unit_start(state["started"])
                    state["started"] += 1
                unit_convert(state["conv"])
                state["conv"] += 1
                if state["started"] < n_units:
                    unit_start(state["started"])
                    state["started"] += 1

        pump(upc * 2)

        for t in range(n_steps):
            s, c = t // n_chunks, t % n_chunks
            if c == 0 and s > 0:
                recv = pltpu.make_async_remote_copy(
                    src_ref=xb_ref.at[pl.ds(0, mper), :],
                    dst_ref=recv_ref.at[s - 1],
                    send_sem=send_sems.at[s - 1],
                    recv_sem=recv_sems.at[s - 1],
                    device_id=((my + s) % N_DEV,),
                    device_id_type=pl.DeviceIdType.MESH,
                )
                recv.wait_recv()
            if s == 0:
                a_blk = xb_ref[pl.ds(my * mper, mper), :]
            else:
                a_blk = recv_ref[s - 1]
            contrib = jnp.dot(
                a_blk, wb_ref[t % RING], preferred_element_type=jnp.float32
            )
            if s == 0:
                acc_ref[:, pl.ds(c * NCHUNK, NCHUNK)] = contrib
            else:
                acc_ref[:, pl.ds(c * NCHUNK, NCHUNK)] += contrib
            if s == N_DEV - 1:
                acc_ref[:, pl.ds(c * NCHUNK, NCHUNK)] = jax.nn.gelu(
                    acc_ref[:, pl.ds(c * NCHUNK, NCHUNK)], approximate=True
                )
                pltpu.make_async_copy(
                    acc_ref.at[:, pl.ds(c * NCHUNK, NCHUNK)],
                    out_ref.at[:, pl.ds(c * NCHUNK, NCHUNK)],
                    osems.at[c],
                ).start()
            pump(upc * (t + 3))

        for c in range(n_chunks):
            pltpu.make_async_copy(
                acc_ref.at[:, pl.ds(c * NCHUNK, NCHUNK)],
                out_ref.at[:, pl.ds(c * NCHUNK, NCHUNK)],
                osems.at[c],
            ).wait()

        for rdma in sends:
            rdma.wait_send()

        @functools.partial(pl.run_scoped, exit_sem=pltpu.SemaphoreType.REGULAR)
        def _(exit_sem):
            for p in range(1, N_DEV):
                pl.semaphore_signal(
                    exit_sem,
                    inc=1,
                    device_id=((my + p) % N_DEV,),
                    device_id_type=pl.DeviceIdType.MESH,
                )
            pl.semaphore_wait(exit_sem, N_DEV - 1)

    return pl.pallas_call(
        body,
        out_shape=jax.ShapeDtypeStruct((mper, n), jnp.float32),
        in_specs=[
            pl.BlockSpec(memory_space=pltpu.MemorySpace.VMEM),
            pl.BlockSpec(memory_space=pltpu.MemorySpace.HBM),
        ],
        out_specs=pl.BlockSpec(memory_space=pltpu.MemorySpace.HBM),
        scratch_shapes=[
            pltpu.VMEM((m_all, mper), jnp.bfloat16),
            pltpu.VMEM((N_DEV - 1, mper, mper), jnp.bfloat16),
            pltpu.VMEM((2, mper, NUNIT), jnp.float32),
            pltpu.VMEM((RING, mper, NCHUNK), jnp.bfloat16),
            pltpu.VMEM((mper, n), jnp.float32),
            pltpu.SemaphoreType.DMA((N_DEV - 1,)),
            pltpu.SemaphoreType.DMA((N_DEV - 1,)),
            pltpu.SemaphoreType.DMA((2,)),
            pltpu.SemaphoreType.DMA((2,)),
        ],
        compiler_params=pltpu.CompilerParams(
            collective_id=0,
            vmem_limit_bytes=100 * 1024 * 1024,
        ),
    )(x, w_mat)
